# Initial kernel scaffold; baseline (speedup 1.0000x reference)
import functools

import jax
import jax.numpy as jnp
from jax import lax
from jax.experimental import pallas as pl
from jax.experimental.pallas import tpu as pltpu, tpu_sc as plsc

_N_USER = 25000
_N_NODES = 50000
_D = 64
_HALF = 25000
_HALF_PAD = 25024
_N_PAD = 2 * _HALF_PAD
_GAP = _HALF_PAD - _HALF
_DUMMY = _HALF

_E = 800000
_SUP = 2048
_CHUNK = 128
_N_TILES = 16
_E_PAD = 819200
_EPT = _E_PAD // _N_TILES
_NSUP = _EPT // _SUP
_ROWS_PER_TILE = _HALF_PAD // _N_TILES


def _layer_body(ego_hbm, src_hbm, dst_hbm, val_hbm, zero_hbm, out_hbm,
                src_sb, dst_sb, val_sb, dst2d, rows, acc, sem):
    c = lax.axis_index("c")
    s = lax.axis_index("s")

    pltpu.sync_copy(zero_hbm, acc.at[pl.ds(s * _ROWS_PER_TILE, _ROWS_PER_TILE)])
    plsc.subcore_barrier()

    dst_base = c * _HALF

    def sup_body(sp, _):
        off = s * _EPT + sp * _SUP
        pltpu.sync_copy(src_hbm.at[pl.ds(off, _SUP)], src_sb)
        pltpu.sync_copy(dst_hbm.at[pl.ds(off, _SUP)], dst_sb)
        pltpu.sync_copy(val_hbm.at[pl.ds(off, _SUP)], val_sb)

        def remap(v, _):
            s16 = src_sb[pl.ds(v * 16, 16)]
            s16 = s16 + jnp.where(s16 >= _HALF, _GAP, 0)
            src_sb[pl.ds(v * 16, 16)] = s16
            d16 = dst_sb[pl.ds(v * 16, 16)] - dst_base
            ok = (d16 >= 0) & (d16 < _HALF)
            d16 = jnp.where(ok, d16, _DUMMY)
            dst2d[v // 8, pl.ds((v % 8) * 16, 16)] = d16
            return 0

        lax.fori_loop(0, _SUP // 16, remap, 0)

        def chunk(b, _):
            pltpu.async_copy(
                ego_hbm.at[src_sb.at[pl.ds(b * _CHUNK, _CHUNK)]], rows, sem
            ).wait()

            def scale(e, _):
                idx = jnp.full((16,), b * _CHUNK + e, dtype=jnp.int32)
                vsp = plsc.load_gather(val_sb, [idx])
                for q in range(_D // 16):
                    rows[e, pl.ds(q * 16, 16)] = rows[e, pl.ds(q * 16, 16)] * vsp
                return 0

            lax.fori_loop(0, _CHUNK, scale, 0)

            pltpu.sync_copy(rows, acc.at[dst2d.at[b]], add=True)
            return 0

        lax.fori_loop(0, _SUP // _CHUNK, chunk, 0)
        return 0

    lax.fori_loop(0, _NSUP, sup_body, 0)

    plsc.subcore_barrier()
    pltpu.sync_copy(
        acc.at[pl.ds(s * _ROWS_PER_TILE, _ROWS_PER_TILE)],
        out_hbm.at[pl.ds(c * _HALF_PAD + s * _ROWS_PER_TILE, _ROWS_PER_TILE)],
    )


@functools.partial(
    pl.kernel,
    mesh=plsc.VectorSubcoreMesh(core_axis_name="c", subcore_axis_name="s"),
    out_type=jax.ShapeDtypeStruct((_N_PAD, _D), jnp.float32),
    scratch_types=[
        pltpu.VMEM((_SUP,), jnp.int32),
        pltpu.VMEM((_SUP,), jnp.int32),
        pltpu.VMEM((_SUP,), jnp.float32),
        pltpu.VMEM((_SUP // _CHUNK, _CHUNK), jnp.int32),
        pltpu.VMEM((_CHUNK, _D), jnp.float32),
        pltpu.VMEM_SHARED((_HALF_PAD, _D), jnp.float32),
        pltpu.SemaphoreType.DMA,
    ],
)
def _propagate_layer(ego_hbm, src_hbm, dst_hbm, val_hbm, zero_hbm, out_hbm,
                     src_sb, dst_sb, val_sb, dst2d, rows, acc, sem):
    _layer_body(ego_hbm, src_hbm, dst_hbm, val_hbm, zero_hbm, out_hbm,
                src_sb, dst_sb, val_sb, dst2d, rows, acc, sem)


def kernel(user_emb, item_emb, adj_indices, adj_values):
    src = adj_indices[0].astype(jnp.int32)
    dst = adj_indices[1].astype(jnp.int32)
    vals = adj_values.astype(jnp.float32)

    pad = _E_PAD - _E
    src_p = jnp.concatenate([src, jnp.zeros((pad,), jnp.int32)])
    dst_p = jnp.concatenate([dst, jnp.full((pad,), -1, jnp.int32)])
    val_p = jnp.concatenate([vals, jnp.zeros((pad,), jnp.float32)])

    ego0 = jnp.zeros((_N_PAD, _D), jnp.float32)
    ego0 = ego0.at[0:_N_USER].set(user_emb)
    ego0 = ego0.at[_HALF_PAD:_HALF_PAD + _N_NODES - _N_USER].set(item_emb)

    zeros = jnp.zeros((_ROWS_PER_TILE, _D), jnp.float32)

    egos = []
    ego = ego0
    for _ in range(3):
        ego = _propagate_layer(ego, src_p, dst_p, val_p, zeros)
        egos.append(ego)

    user_all = jnp.stack(
        [user_emb] + [e[0:_N_USER] for e in egos], axis=1)
    item_all = jnp.stack(
        [item_emb] + [e[_HALF_PAD:_HALF_PAD + _N_NODES - _N_USER] for e in egos],
        axis=1)
    return (user_all, item_all)


# speedup vs baseline: 2.6958x; 2.6958x over previous
import functools

import jax
import jax.numpy as jnp
from jax import lax
from jax.experimental import pallas as pl
from jax.experimental.pallas import tpu as pltpu, tpu_sc as plsc

_N_USER = 25000
_N_NODES = 50000
_D = 64
_HALF = 25000
_HALF_PAD = 25088
_N_PAD = 2 * _HALF_PAD
_GAP = _HALF_PAD - _HALF
_DUMMY = _HALF

_E = 800000
_SUP = 2048
_CHUNK = 128
_N_TILES = 16
_E_PAD = 819200
_EPT = _E_PAD // _N_TILES
_NSUP = _EPT // _SUP
_ROWS_PER_TILE = _HALF_PAD // _N_TILES


def _layer_body(ego_hbm, src_hbm, dst_hbm, val_hbm, zero_hbm, out_hbm,
                src_sb, dst_sb, val_sb, dst2d, rows, acc, sem):
    c = lax.axis_index("c")
    s = lax.axis_index("s")

    pltpu.sync_copy(zero_hbm, acc.at[pl.ds(s * _ROWS_PER_TILE, _ROWS_PER_TILE)])
    plsc.subcore_barrier()

    dst_base = c * _HALF

    def sup_body(sp, _):
        off = s * _EPT + sp * _SUP
        pltpu.sync_copy(src_hbm.at[pl.ds(off, _SUP)], src_sb)
        pltpu.sync_copy(dst_hbm.at[pl.ds(off, _SUP)], dst_sb)
        pltpu.sync_copy(val_hbm.at[pl.ds(off, _SUP)], val_sb)

        def remap(v, _):
            s16 = src_sb[pl.ds(v * 16, 16)]
            s16 = s16 + jnp.where(s16 >= _HALF, _GAP, 0)
            src_sb[pl.ds(v * 16, 16)] = s16
            d16 = dst_sb[pl.ds(v * 16, 16)] - dst_base
            ok = (d16 >= 0) & (d16 < _HALF)
            d16 = jnp.where(ok, d16, _DUMMY)
            dst2d[v // 8, pl.ds((v % 8) * 16, 16)] = d16
            return 0

        lax.fori_loop(0, _SUP // 16, remap, 0)

        def chunk(b, _):
            pltpu.async_copy(
                ego_hbm.at[src_sb.at[pl.ds(b * _CHUNK, _CHUNK)]], rows, sem
            ).wait()

            def scale16(g, _):
                v16 = val_sb[pl.ds(b * _CHUNK + g * 16, 16)]
                for j in range(16):
                    vsp = jnp.broadcast_to(v16[j], (16,))
                    e = g * 16 + j
                    for q in range(_D // 16):
                        rows[e, pl.ds(q * 16, 16)] = (
                            rows[e, pl.ds(q * 16, 16)] * vsp)
                return 0

            lax.fori_loop(0, _CHUNK // 16, scale16, 0)

            pltpu.sync_copy(rows, acc.at[dst2d.at[b]], add=True)
            return 0

        lax.fori_loop(0, _SUP // _CHUNK, chunk, 0)
        return 0

    lax.fori_loop(0, _NSUP, sup_body, 0)

    plsc.subcore_barrier()
    pltpu.sync_copy(
        acc.at[pl.ds(s * _ROWS_PER_TILE, _ROWS_PER_TILE)],
        out_hbm.at[pl.ds(c * _HALF_PAD + s * _ROWS_PER_TILE, _ROWS_PER_TILE)],
    )


@functools.partial(
    pl.kernel,
    mesh=plsc.VectorSubcoreMesh(core_axis_name="c", subcore_axis_name="s"),
    out_type=jax.ShapeDtypeStruct((_N_PAD, _D), jnp.float32),
    scratch_types=[
        pltpu.VMEM((_SUP,), jnp.int32),
        pltpu.VMEM((_SUP,), jnp.int32),
        pltpu.VMEM((_SUP,), jnp.float32),
        pltpu.VMEM((_SUP // _CHUNK, _CHUNK), jnp.int32),
        pltpu.VMEM((_CHUNK, _D), jnp.float32),
        pltpu.VMEM_SHARED((_HALF_PAD, _D), jnp.float32),
        pltpu.SemaphoreType.DMA,
    ],
    compiler_params=pltpu.CompilerParams(use_tc_tiling_on_sc=False),
)
def _propagate_layer(ego_hbm, src_hbm, dst_hbm, val_hbm, zero_hbm, out_hbm,
                     src_sb, dst_sb, val_sb, dst2d, rows, acc, sem):
    _layer_body(ego_hbm, src_hbm, dst_hbm, val_hbm, zero_hbm, out_hbm,
                src_sb, dst_sb, val_sb, dst2d, rows, acc, sem)


def kernel(user_emb, item_emb, adj_indices, adj_values):
    src = adj_indices[0].astype(jnp.int32)
    dst = adj_indices[1].astype(jnp.int32)
    vals = adj_values.astype(jnp.float32)

    pad = _E_PAD - _E
    src_p = jnp.concatenate([src, jnp.zeros((pad,), jnp.int32)])
    dst_p = jnp.concatenate([dst, jnp.full((pad,), -1, jnp.int32)])
    val_p = jnp.concatenate([vals, jnp.zeros((pad,), jnp.float32)])

    ego0 = jnp.zeros((_N_PAD, _D), jnp.float32)
    ego0 = ego0.at[0:_N_USER].set(user_emb)
    ego0 = ego0.at[_HALF_PAD:_HALF_PAD + _N_NODES - _N_USER].set(item_emb)

    zeros = jnp.zeros((_ROWS_PER_TILE, _D), jnp.float32)

    egos = []
    ego = ego0
    for _ in range(3):
        ego = _propagate_layer(ego, src_p, dst_p, val_p, zeros)
        egos.append(ego)

    user_all = jnp.stack(
        [user_emb] + [e[0:_N_USER] for e in egos], axis=1)
    item_all = jnp.stack(
        [item_emb] + [e[_HALF_PAD:_HALF_PAD + _N_NODES - _N_USER] for e in egos],
        axis=1)
    return (user_all, item_all)


# speedup vs baseline: 6.6554x; 2.4688x over previous
import functools

import jax
import jax.numpy as jnp
from jax import lax
from jax.experimental import pallas as pl
from jax.experimental.pallas import tpu as pltpu, tpu_sc as plsc

_N_USER = 25000
_N_NODES = 50000
_D = 64
_DH = 32
_N2 = 50048
_SINK = 50000

_E = 800000
_SUP = 2048
_CHUNK = 128
_NCH = _SUP // _CHUNK
_N_TILES = 16
_E_PAD = 819200
_EPT = _E_PAD // _N_TILES
_NSUP = _EPT // _SUP
_RPT = _N2 // _N_TILES
_RPT_LAST = _N_NODES - 15 * _RPT


def _layer_body(ego_hbm, src2_hbm, dst2_hbm, val_hbm, zero_hbm, out_hbm,
                src_sb, val_sb, dst2d, rows4, acc, *sems):
    sem_g, sem_s = sems[:4], sems[4:]
    c = lax.axis_index("c")
    s = lax.axis_index("s")

    pltpu.sync_copy(zero_hbm, acc.at[pl.ds(s * _RPT, _RPT)])
    plsc.subcore_barrier()

    def sup_body(sp, _):
        off = s * _EPT + sp * _SUP
        pltpu.sync_copy(src2_hbm.at[c, pl.ds(off, _SUP)], src_sb)
        for k in range(3):
            pltpu.async_copy(
                ego_hbm.at[src_sb.at[pl.ds(k * _CHUNK, _CHUNK)]],
                rows4.at[k], sem_g[k])
        pltpu.sync_copy(dst2_hbm.at[pl.ds(off // _CHUNK, _NCH)], dst2d)
        pltpu.sync_copy(val_hbm.at[pl.ds(off, _SUP)], val_sb)

        def chunk4(b4, _):
            for k in range(4):
                b = b4 * 4 + k
                pltpu.make_async_copy(
                    ego_hbm.at[pl.ds(0, _CHUNK)], rows4.at[k], sem_g[k]).wait()

                @pl.when(b >= 1)
                def _():
                    pltpu.make_async_copy(
                        ego_hbm.at[pl.ds(0, _CHUNK)], rows4.at[(k + 3) % 4],
                        sem_s[(k + 3) % 4]).wait()

                @pl.when(b + 3 < _NCH)
                def _():
                    pltpu.async_copy(
                        ego_hbm.at[src_sb.at[pl.ds((b + 3) * _CHUNK, _CHUNK)]],
                        rows4.at[(k + 3) % 4], sem_g[(k + 3) % 4])

                def scale16(g, _):
                    v16 = val_sb[pl.ds(b * _CHUNK + g * 16, 16)]
                    for j in range(16):
                        vsp = jnp.broadcast_to(v16[j], (16,))
                        e = g * 16 + j
                        for q in range(_DH // 16):
                            rows4[k, e, pl.ds(q * 16, 16)] = (
                                rows4[k, e, pl.ds(q * 16, 16)] * vsp)
                    return 0

                lax.fori_loop(0, _CHUNK // 16, scale16, 0)

                pltpu.async_copy(rows4.at[k], acc.at[dst2d.at[b]],
                                 sem_s[k], add=True)
            return 0

        lax.fori_loop(0, _NCH // 4, chunk4, 0)
        pltpu.make_async_copy(
            ego_hbm.at[pl.ds(0, _CHUNK)], rows4.at[3], sem_s[3]).wait()
        return 0

    lax.fori_loop(0, _NSUP, sup_body, 0)

    plsc.subcore_barrier()
    @pl.when(s < _N_TILES - 1)
    def _():
        pltpu.sync_copy(acc.at[pl.ds(s * _RPT, _RPT)],
                        out_hbm.at[pl.ds(c * _N2 + s * _RPT, _RPT)])

    @pl.when(s == _N_TILES - 1)
    def _():
        pltpu.sync_copy(
            acc.at[pl.ds((_N_TILES - 1) * _RPT, _RPT_LAST)],
            out_hbm.at[pl.ds(c * _N2 + (_N_TILES - 1) * _RPT, _RPT_LAST)])


@functools.partial(
    pl.kernel,
    mesh=plsc.VectorSubcoreMesh(core_axis_name="c", subcore_axis_name="s"),
    out_type=jax.ShapeDtypeStruct((2 * _N2, _DH), jnp.float32),
    scratch_types=[
        pltpu.VMEM((_SUP,), jnp.int32),
        pltpu.VMEM((_SUP,), jnp.float32),
        pltpu.VMEM((_NCH, _CHUNK), jnp.int32),
        pltpu.VMEM((4, _CHUNK, _DH), jnp.float32),
        pltpu.VMEM_SHARED((_N2, _DH), jnp.float32),
        pltpu.SemaphoreType.DMA,
        pltpu.SemaphoreType.DMA,
        pltpu.SemaphoreType.DMA,
        pltpu.SemaphoreType.DMA,
        pltpu.SemaphoreType.DMA,
        pltpu.SemaphoreType.DMA,
        pltpu.SemaphoreType.DMA,
        pltpu.SemaphoreType.DMA,
    ],
    compiler_params=pltpu.CompilerParams(use_tc_tiling_on_sc=False),
)
def _propagate_layer(ego_hbm, src2_hbm, dst2_hbm, val_hbm, zero_hbm, out_hbm,
                     src_sb, val_sb, dst2d, rows4, acc, *sems):
    _layer_body(ego_hbm, src2_hbm, dst2_hbm, val_hbm, zero_hbm, out_hbm,
                src_sb, val_sb, dst2d, rows4, acc, *sems)


def kernel(user_emb, item_emb, adj_indices, adj_values):
    src = adj_indices[0].astype(jnp.int32)
    dst = adj_indices[1].astype(jnp.int32)
    vals = adj_values.astype(jnp.float32)

    pad = _E_PAD - _E
    src_p = jnp.concatenate([src, jnp.zeros((pad,), jnp.int32)])
    dst_p = jnp.concatenate([dst, jnp.full((pad,), _SINK, jnp.int32)])
    val_p = jnp.concatenate([vals, jnp.zeros((pad,), jnp.float32)])

    src2 = jnp.stack([src_p, src_p + _N2])
    dst2 = dst_p.reshape(_E_PAD // _CHUNK, _CHUNK)

    ego_full = jnp.concatenate([user_emb, item_emb], axis=0)
    ego = jnp.zeros((2 * _N2, _DH), jnp.float32)
    ego = ego.at[0:_N_NODES].set(ego_full[:, :_DH])
    ego = ego.at[_N2:_N2 + _N_NODES].set(ego_full[:, _DH:])

    zeros = jnp.zeros((_RPT, _DH), jnp.float32)

    egos = []
    for _ in range(3):
        ego = _propagate_layer(ego, src2, dst2, val_p, zeros)
        egos.append(jnp.concatenate(
            [ego[:_N_NODES], ego[_N2:_N2 + _N_NODES]], axis=1))

    user_all = jnp.stack([user_emb] + [e[:_N_USER] for e in egos], axis=1)
    item_all = jnp.stack([item_emb] + [e[_N_USER:] for e in egos], axis=1)
    return (user_all, item_all)
